# Initial kernel scaffold; baseline (speedup 1.0000x reference)
import jax
import jax.numpy as jnp
from jax.experimental import pallas as pl


def kernel(hidden_states, gate_w, w1, w2, w3):
    raise NotImplementedError("write your pallas kernel here")


# speedup vs baseline: 1.5035x; 1.5035x over previous
import functools

import jax
import jax.numpy as jnp
from jax.experimental import pallas as pl
from jax.experimental.pallas import tpu as pltpu

NUM_EXPERTS = 8
TOP_K = 2
HIDDEN = 1024
FFN = 3584
TOKENS = 32

FC = 512
NF = FFN // FC


def _moe_kernel(x_ref, gate_ref, w1_ref, w2_ref, w3_ref, out_ref, wt_scr):
    e = pl.program_id(0)
    j = pl.program_id(1)

    x = x_ref[:, :]

    @pl.when((e == 0) & (j == 0))
    def _init():
        logits = jax.lax.dot_general(
            x, gate_ref[:, :], (((1,), (1,)), ((), ())),
            preferred_element_type=jnp.float32)
        probs = jax.nn.softmax(logits, axis=1)
        iota = jax.lax.broadcasted_iota(jnp.int32, (TOKENS, NUM_EXPERTS), 1)
        m1 = jnp.max(probs, axis=1, keepdims=True)
        i1 = jnp.min(jnp.where(probs == m1, iota, NUM_EXPERTS), axis=1,
                     keepdims=True)
        masked = jnp.where(iota == i1, -1.0, probs)
        m2 = jnp.max(masked, axis=1, keepdims=True)
        i2 = jnp.min(jnp.where(masked == m2, iota, NUM_EXPERTS), axis=1,
                     keepdims=True)
        top2 = (iota == i1) | (iota == i2)
        wt_scr[:, :] = jnp.where(top2, probs / (m1 + m2), 0.0)
        out_ref[:, :] = jnp.zeros_like(out_ref)

    lane = jax.lax.broadcasted_iota(jnp.int32, (TOKENS, NUM_EXPERTS), 1)
    wt = jnp.sum(jnp.where(lane == e, wt_scr[:, :], 0.0), axis=1,
                 keepdims=True)

    h = jax.lax.dot_general(x, w1_ref[0], (((1,), (1,)), ((), ())),
                            preferred_element_type=jnp.float32)
    g = jax.lax.dot_general(x, w3_ref[0], (((1,), (1,)), ((), ())),
                            preferred_element_type=jnp.float32)
    act = (h * jax.lax.logistic(h)) * g
    partial = jax.lax.dot_general(act, w2_ref[0], (((1,), (1,)), ((), ())),
                                  preferred_element_type=jnp.float32)
    out_ref[:, :] += wt * partial


@functools.partial(jax.jit, static_argnames=())
def kernel(hidden_states, gate_w, w1, w2, w3):
    grid = (NUM_EXPERTS, NF)
    return pl.pallas_call(
        _moe_kernel,
        grid=grid,
        in_specs=[
            pl.BlockSpec((TOKENS, HIDDEN), lambda e, j: (0, 0)),
            pl.BlockSpec((NUM_EXPERTS, HIDDEN), lambda e, j: (0, 0)),
            pl.BlockSpec((1, FC, HIDDEN), lambda e, j: (e, j, 0)),
            pl.BlockSpec((1, HIDDEN, FC), lambda e, j: (e, 0, j)),
            pl.BlockSpec((1, FC, HIDDEN), lambda e, j: (e, j, 0)),
        ],
        out_specs=pl.BlockSpec((TOKENS, HIDDEN), lambda e, j: (0, 0)),
        out_shape=jax.ShapeDtypeStruct((TOKENS, HIDDEN), jnp.float32),
        scratch_shapes=[pltpu.VMEM((TOKENS, NUM_EXPERTS), jnp.float32)],
        compiler_params=pltpu.CompilerParams(
            dimension_semantics=("arbitrary", "arbitrary"),
        ),
    )(hidden_states, gate_w, w1, w2, w3)


# speedup vs baseline: 1.6551x; 1.1008x over previous
import functools

import jax
import jax.numpy as jnp
from jax.experimental import pallas as pl
from jax.experimental.pallas import tpu as pltpu

NUM_EXPERTS = 8
TOP_K = 2
HIDDEN = 1024
FFN = 3584
TOKENS = 32

FC = 896
NF = FFN // FC


def _moe_kernel(x_ref, gate_ref, w1_ref, w2_ref, w3_ref, out_ref, wt_scr):
    e = pl.program_id(0)
    j = pl.program_id(1)

    x = x_ref[:, :]

    @pl.when((e == 0) & (j == 0))
    def _init():
        logits = jax.lax.dot_general(
            x, gate_ref[:, :], (((1,), (1,)), ((), ())),
            preferred_element_type=jnp.float32)
        probs = jax.nn.softmax(logits, axis=1)
        iota = jax.lax.broadcasted_iota(jnp.int32, (TOKENS, NUM_EXPERTS), 1)
        m1 = jnp.max(probs, axis=1, keepdims=True)
        i1 = jnp.min(jnp.where(probs == m1, iota, NUM_EXPERTS), axis=1,
                     keepdims=True)
        masked = jnp.where(iota == i1, -1.0, probs)
        m2 = jnp.max(masked, axis=1, keepdims=True)
        i2 = jnp.min(jnp.where(masked == m2, iota, NUM_EXPERTS), axis=1,
                     keepdims=True)
        top2 = (iota == i1) | (iota == i2)
        wt_scr[:, :] = jnp.where(top2, probs / (m1 + m2), 0.0)
        out_ref[:, :] = jnp.zeros_like(out_ref)

    lane = jax.lax.broadcasted_iota(jnp.int32, (TOKENS, NUM_EXPERTS), 1)
    wt = jnp.sum(jnp.where(lane == e, wt_scr[:, :], 0.0), axis=1,
                 keepdims=True)

    h = jax.lax.dot_general(x, w1_ref[0], (((1,), (1,)), ((), ())),
                            preferred_element_type=jnp.float32)
    g = jax.lax.dot_general(x, w3_ref[0], (((1,), (1,)), ((), ())),
                            preferred_element_type=jnp.float32)
    act = (h * jax.lax.logistic(h)) * g
    partial = jax.lax.dot_general(act, w2_ref[0], (((1,), (1,)), ((), ())),
                                  preferred_element_type=jnp.float32)
    out_ref[:, :] += wt * partial


@functools.partial(jax.jit, static_argnames=())
def kernel(hidden_states, gate_w, w1, w2, w3):
    grid = (NUM_EXPERTS, NF)
    return pl.pallas_call(
        _moe_kernel,
        grid=grid,
        in_specs=[
            pl.BlockSpec((TOKENS, HIDDEN), lambda e, j: (0, 0)),
            pl.BlockSpec((NUM_EXPERTS, HIDDEN), lambda e, j: (0, 0)),
            pl.BlockSpec((1, FC, HIDDEN), lambda e, j: (e, j, 0)),
            pl.BlockSpec((1, HIDDEN, FC), lambda e, j: (e, 0, j)),
            pl.BlockSpec((1, FC, HIDDEN), lambda e, j: (e, j, 0)),
        ],
        out_specs=pl.BlockSpec((TOKENS, HIDDEN), lambda e, j: (0, 0)),
        out_shape=jax.ShapeDtypeStruct((TOKENS, HIDDEN), jnp.float32),
        scratch_shapes=[pltpu.VMEM((TOKENS, NUM_EXPERTS), jnp.float32)],
        compiler_params=pltpu.CompilerParams(
            dimension_semantics=("arbitrary", "arbitrary"),
        ),
    )(hidden_states, gate_w, w1, w2, w3)


# speedup vs baseline: 1.6662x; 1.0067x over previous
import functools

import jax
import jax.numpy as jnp
from jax.experimental import pallas as pl
from jax.experimental.pallas import tpu as pltpu

NUM_EXPERTS = 8
TOP_K = 2
HIDDEN = 1024
FFN = 3584
TOKENS = 32

FC = 896
NF = FFN // FC


def _moe_kernel(x_ref, gate_ref, w1_ref, w2_ref, w3_ref, out_ref, wt_scr):
    e = pl.program_id(0)
    j = pl.program_id(1)

    x = x_ref[:, :]

    @pl.when((e == 0) & (j == 0))
    def _init():
        logits = jax.lax.dot_general(
            x, gate_ref[:, :], (((1,), (1,)), ((), ())),
            preferred_element_type=jnp.float32)
        probs = jax.nn.softmax(logits, axis=1)
        iota = jax.lax.broadcasted_iota(jnp.int32, (TOKENS, NUM_EXPERTS), 1)
        m1 = jnp.max(probs, axis=1, keepdims=True)
        i1 = jnp.min(jnp.where(probs == m1, iota, NUM_EXPERTS), axis=1,
                     keepdims=True)
        masked = jnp.where(iota == i1, -1.0, probs)
        m2 = jnp.max(masked, axis=1, keepdims=True)
        i2 = jnp.min(jnp.where(masked == m2, iota, NUM_EXPERTS), axis=1,
                     keepdims=True)
        top2 = (iota == i1) | (iota == i2)
        wt_scr[:, :] = jnp.where(top2, probs / (m1 + m2), 0.0)
        out_ref[:, :] = jnp.zeros_like(out_ref)

    lane = jax.lax.broadcasted_iota(jnp.int32, (TOKENS, NUM_EXPERTS), 1)
    wt = jnp.sum(jnp.where(lane == e, wt_scr[:, :], 0.0), axis=1,
                 keepdims=True)

    xb = x.astype(jnp.bfloat16)
    h = jax.lax.dot_general(xb, w1_ref[0].astype(jnp.bfloat16),
                            (((1,), (1,)), ((), ())),
                            preferred_element_type=jnp.float32)
    g = jax.lax.dot_general(xb, w3_ref[0].astype(jnp.bfloat16),
                            (((1,), (1,)), ((), ())),
                            preferred_element_type=jnp.float32)
    act = (h * jax.lax.logistic(h)) * g
    partial = jax.lax.dot_general(act.astype(jnp.bfloat16),
                                  w2_ref[0].astype(jnp.bfloat16),
                                  (((1,), (1,)), ((), ())),
                                  preferred_element_type=jnp.float32)
    out_ref[:, :] += wt * partial


@functools.partial(jax.jit, static_argnames=())
def kernel(hidden_states, gate_w, w1, w2, w3):
    grid = (NUM_EXPERTS, NF)
    return pl.pallas_call(
        _moe_kernel,
        grid=grid,
        in_specs=[
            pl.BlockSpec((TOKENS, HIDDEN), lambda e, j: (0, 0)),
            pl.BlockSpec((NUM_EXPERTS, HIDDEN), lambda e, j: (0, 0)),
            pl.BlockSpec((1, FC, HIDDEN), lambda e, j: (e, j, 0)),
            pl.BlockSpec((1, HIDDEN, FC), lambda e, j: (e, 0, j)),
            pl.BlockSpec((1, FC, HIDDEN), lambda e, j: (e, j, 0)),
        ],
        out_specs=pl.BlockSpec((TOKENS, HIDDEN), lambda e, j: (0, 0)),
        out_shape=jax.ShapeDtypeStruct((TOKENS, HIDDEN), jnp.float32),
        scratch_shapes=[pltpu.VMEM((TOKENS, NUM_EXPERTS), jnp.float32)],
        compiler_params=pltpu.CompilerParams(
            dimension_semantics=("arbitrary", "arbitrary"),
        ),
    )(hidden_states, gate_w, w1, w2, w3)
